# baseline (speedup 1.0000x reference)
import functools

import jax
import jax.numpy as jnp
import numpy as np
from jax.experimental import pallas as pl
from jax.experimental.pallas import tpu as pltpu
from jax.experimental.pallas import tpu_sc as plsc

_N_NODES = 10000
_N_EDGES = 160000
_DIM = 128
_HEADS = 8
_DNP = _DIM // _HEADS
_TEMP = float(np.sqrt(_DNP))
_NP_PAD = 10240

_BE = 2000
_BN = 1024


_GW = 200


def _sc_gather(table, idx):
    n_orig = idx.shape[0]
    chunk = _GW * 32
    n = ((n_orig + chunk - 1) // chunk) * chunk
    if n != n_orig:
        idx = jnp.pad(idx, (0, n - n_orig))
    mesh = plsc.VectorSubcoreMesh(core_axis_name="core",
                                  subcore_axis_name="subcore")

    share = n // 32
    ncols = table.shape[1]

    @functools.partial(
        pl.kernel,
        out_type=jax.ShapeDtypeStruct((n, ncols), table.dtype),
        mesh=mesh,
        scratch_types=[pltpu.VMEM((_GW,), jnp.int32),
                       pltpu.VMEM((_GW, ncols), jnp.float32),
                       pltpu.SemaphoreType.DMA])
    def k(tab_hbm, i_hbm, o_hbm, idx_v, rows_v, sem):
        cid = jax.lax.axis_index("core")
        sid = jax.lax.axis_index("subcore")
        base = (cid * 16 + sid) * share

        @pl.loop(0, share, step=_GW)
        def _(off):
            pltpu.sync_copy(i_hbm.at[pl.ds(base + off, _GW)], idx_v)
            pltpu.async_copy(tab_hbm.at[idx_v], rows_v, sem).wait()
            pltpu.sync_copy(rows_v, o_hbm.at[pl.ds(base + off, _GW)])

    out = k(table, idx)
    return out[:n_orig] if n != n_orig else out


_RK = 200
_NPW = _NP_PAD // 32


def _sc_segmax(values, order_pad, ids_pad, bounds):
    mesh = plsc.VectorSubcoreMesh(core_axis_name="core",
                                  subcore_axis_name="subcore")
    neg = jnp.float32(-jnp.inf)

    @functools.partial(
        pl.kernel,
        out_type=jax.ShapeDtypeStruct((_NP_PAD, _DIM), jnp.float32),
        mesh=mesh,
        scratch_types=[pltpu.VMEM((_NPW, _DIM), jnp.float32),
                       pltpu.VMEM((_RK,), jnp.int32),
                       pltpu.VMEM((_RK, _DIM), jnp.float32),
                       pltpu.VMEM((_RK,), jnp.int32),
                       pltpu.VMEM((16,), jnp.int32),
                       pltpu.SemaphoreType.DMA])
    def k(val_hbm, ord_hbm, ids_hbm, bnd_hbm, o_hbm,
          acc_v, oidx_v, rows_v, ids_v, bnd_v, sem):
        cid = jax.lax.axis_index("core")
        sid = jax.lax.axis_index("subcore")
        w = cid * 16 + sid
        node_base = w * _NPW

        pltpu.sync_copy(bnd_hbm.at[w], bnd_v)
        bv = bnd_v[...]
        lo = bv[0]
        hi = bv[1]

        @pl.loop(0, _NPW)
        def _(r):
            @pl.loop(0, _DIM, step=16)
            def _(j):
                acc_v[r, pl.ds(j, 16)] = jnp.full((16,), neg)

        start0 = (lo // 8) * 8
        n_win = jax.lax.div(hi - start0 + _RK - 1, _RK)

        @pl.loop(0, n_win)
        def _(t):
            base = start0 + t * _RK
            pltpu.sync_copy(ord_hbm.at[pl.ds(base, _RK)], oidx_v)
            pltpu.sync_copy(ids_hbm.at[pl.ds(base, _RK)], ids_v)
            pltpu.async_copy(val_hbm.at[oidx_v], rows_v, sem).wait()

            @pl.loop(0, _RK, step=16)
            def _(g):
                vec = ids_v[pl.ds(g, 16)]
                for e0 in range(16):
                    local = vec[e0] - node_base

                    @pl.when(jnp.logical_and(local >= 0, local < _NPW))
                    def _():
                        for j in range(0, _DIM, 16):
                            acc_v[local, pl.ds(j, 16)] = jnp.maximum(
                                acc_v[local, pl.ds(j, 16)],
                                rows_v[g + e0, pl.ds(j, 16)])

        pltpu.sync_copy(acc_v, o_hbm.at[pl.ds(node_base, _NPW)])

    return k(values, order_pad, ids_pad, bounds)


def _head_max(x):
    m = x
    for k in (8, 16, 32, 64):
        m = jnp.maximum(m, jnp.roll(m, k, axis=-1))
    return m


def _head_sum(x):
    s = x
    for k in (8, 16, 32, 64):
        s = s + jnp.roll(s, k, axis=-1)
    return s


def _edge_kernel(xi_ref, ef_ref, efr_ref, xj_ref, match_ref,
                 w1t_ref, b1_ref, w2t_ref, b2_ref,
                 wqt_ref, bq_ref, wkt_ref, bk_ref, wvt_ref, bv_ref,
                 m1_ref, a1_ref, m2_ref, a2_ref,
                 ue_ref, prob_ref, wgt_ref, *, pre_relu):
    xi = xi_ref[...]
    ef = ef_ref[...]
    efr = efr_ref[...] * match_ref[...]
    xj = xj_ref[...]
    if pre_relu:
        ef = jnp.maximum(ef, 0.0)
        efr = jnp.maximum(efr, 0.0)

    f32 = jnp.float32
    dot = functools.partial(jnp.dot, preferred_element_type=f32)

    w1t = w1t_ref[...]
    h = (dot(xi, w1t[0:128]) + dot(ef, w1t[128:256])
         + dot(efr, w1t[256:384]) + dot(xj, w1t[384:512]) + b1_ref[...])
    h = jnp.maximum(h, 0.0)
    ue = dot(h, w2t_ref[...]) + b2_ref[...]
    ue_ref[...] = ue

    q = dot(xi, wqt_ref[...]) + bq_ref[...]
    kk = dot(ef, wkt_ref[...]) + bk_ref[...]
    v = dot(xj, wvt_ref[...]) + bv_ref[...]

    m1 = m1_ref[...]
    a = dot(q, m1[0:128]) + dot(kk, m1[128:256]) + a1_ref[...]
    a = jnp.maximum(a, 0.0)
    att = dot(a, m2_ref[...]) + a2_ref[...]
    att = att * (1.0 / _TEMP)

    mx = _head_max(att)
    e = jnp.exp(att - mx)
    s = _head_sum(e)
    prob = e / s
    prob_ref[...] = prob
    wgt_ref[...] = prob * v


def _edge_compute(xi, ef, efr, xj, match, wp, pre_relu):
    grid = _N_EDGES // _BE
    bspec_e = pl.BlockSpec((_BE, _DIM), lambda i: (i, 0))
    bspec_m = pl.BlockSpec((_BE, 1), lambda i: (i, 0))

    def wspec(arr):
        return pl.BlockSpec(arr.shape, lambda i: tuple(0 for _ in arr.shape))

    weights = [wp['eu_W1T'], wp['eu_b1'], wp['eu_W2T'], wp['eu_b2'],
               wp['WqT'], wp['bq'], wp['WkT'], wp['bk'], wp['WvT'], wp['bv'],
               wp['M1'], wp['a1'], wp['M2'], wp['a2']]
    out_shape = [jax.ShapeDtypeStruct((_N_EDGES, _DIM), jnp.float32)] * 3
    return pl.pallas_call(
        functools.partial(_edge_kernel, pre_relu=pre_relu),
        grid=grid,
        in_specs=[bspec_e] * 4 + [bspec_m] + [wspec(w) for w in weights],
        out_specs=[bspec_e] * 3,
        out_shape=out_shape,
    )(xi, ef, efr, xj, match, *weights)


def _node_kernel(x_ref, agg_ref, so0_ref, so1_ref, si0_ref, si1_ref,
                 ico_ref, ici_ref, mask_ref,
                 nw1t_ref, nb1_ref, nw2t_ref, nb2_ref, eawt_ref, eab_ref,
                 out_ref):
    f32 = jnp.float32
    dot = functools.partial(jnp.dot, preferred_element_type=f32)
    x = x_ref[...]
    agg = agg_ref[...]
    agg = jnp.where(mask_ref[...] > 0.0, agg, 0.0)
    nw1t = nw1t_ref[...]
    h = dot(x, nw1t[0:128]) + dot(agg, nw1t[128:256]) + nb1_ref[...]
    h = jnp.maximum(h, 0.0)
    un = dot(h, nw2t_ref[...]) + nb2_ref[...]

    mean_out = (so0_ref[...] + so1_ref[...]) * ico_ref[...]
    mean_in = (si0_ref[...] + si1_ref[...]) * ici_ref[...]
    eawt = eawt_ref[...]
    logits = dot(mean_out, eawt[0:128]) + dot(mean_in, eawt[128:256]) + eab_ref[...]
    ea = jax.nn.sigmoid(logits)
    out_ref[...] = jnp.maximum(un, 0.0) * ea


def _node_compute(x_pad, agg, so0, so1, si0, si1, ico, ici, mask, wp):
    grid = _NP_PAD // _BN
    bspec_n = pl.BlockSpec((_BN, _DIM), lambda i: (i, 0))
    bspec_1 = pl.BlockSpec((_BN, 1), lambda i: (i, 0))

    def wspec(arr):
        return pl.BlockSpec(arr.shape, lambda i: tuple(0 for _ in arr.shape))

    weights = [wp['nu_W1T'], wp['nu_b1'], wp['nu_W2T'], wp['nu_b2'],
               wp['ea_WT'], wp['ea_b']]
    return pl.pallas_call(
        _node_kernel,
        grid=grid,
        in_specs=[bspec_n] * 6 + [bspec_1] * 3 + [wspec(w) for w in weights],
        out_specs=bspec_n,
        out_shape=jax.ShapeDtypeStruct((_NP_PAD, _DIM), jnp.float32),
    )(x_pad, agg, so0, so1, si0, si1, ico, ici, mask, *weights)


def _prep_weights(p):
    eye8 = jnp.eye(_HEADS, dtype=jnp.float32)
    return {
        'eu_W1T': p['eu_W1'].T, 'eu_b1': p['eu_b1'][None, :],
        'eu_W2T': p['eu_W2'].T, 'eu_b2': p['eu_b2'][None, :],
        'WqT': p['Wq'].T, 'bq': p['bq'][None, :],
        'WkT': p['Wk'].T, 'bk': p['bk'][None, :],
        'WvT': p['Wv'].T, 'bv': p['bv'][None, :],
        'M1': jnp.kron(p['att_W1'].T, eye8),
        'a1': jnp.repeat(p['att_b1'], _HEADS)[None, :],
        'M2': jnp.kron(p['att_W2'].T, eye8),
        'a2': jnp.repeat(p['att_b2'], _HEADS)[None, :],
        'nu_W1T': p['nu_W1'].T, 'nu_b1': p['nu_b1'][None, :],
        'nu_W2T': p['nu_W2'].T, 'nu_b2': p['nu_b2'][None, :],
        'ea_WT': p['ea_W'].T, 'ea_b': p['ea_b'][None, :],
    }


def kernel(x, edge_feature, edge_index, params):
    row, col = edge_index[0], edge_index[1]
    E = _N_EDGES

    keys = row * _N_NODES + col
    rev = col * _N_NODES + row
    order = jnp.argsort(keys)
    sk = keys[order]
    pos = jnp.clip(jnp.searchsorted(sk, rev, method='sort'), 0, E - 1)
    match = sk[pos] == rev
    rev_idx = jnp.where(match, order[pos], jnp.arange(E, dtype=pos.dtype))
    match_f = match.astype(jnp.float32)[:, None]

    ones = jnp.ones((E,), jnp.float32)
    cnt_out = jax.ops.segment_sum(ones, row, num_segments=_N_NODES)
    cnt_in = jax.ops.segment_sum(ones, col, num_segments=_N_NODES)

    row_sorted = (sk // _N_NODES).astype(jnp.int32)
    bnd = jnp.searchsorted(
        row_sorted, jnp.arange(33, dtype=jnp.int32) * _NPW).astype(jnp.int32)
    bounds = jnp.zeros((32, 16), jnp.int32)
    bounds = bounds.at[:, 0].set(bnd[:32]).at[:, 1].set(bnd[1:33])
    order_pad = jnp.pad(order.astype(jnp.int32), (0, _RK))
    ids_pad = jnp.pad(row_sorted, (0, _RK), constant_values=1 << 20)

    def pad_n(v):
        return jnp.pad(v, ((0, _NP_PAD - _N_NODES), (0, 0)))

    ico = pad_n((1.0 / jnp.maximum(cnt_out, 1.0))[:, None])
    ici = pad_n((1.0 / jnp.maximum(cnt_in, 1.0))[:, None])
    mask_out = pad_n((cnt_out > 0).astype(jnp.float32)[:, None])

    nf, ef = x, edge_feature
    probs = []
    for li, p in enumerate(params):
        wp = _prep_weights(p)
        pre_relu = li > 0
        xi = _sc_gather(nf, row)
        xj = _sc_gather(nf, col)
        efr = _sc_gather(ef, rev_idx)

        ue, prob, wgt = _edge_compute(xi, ef, efr, xj, match_f, wp,
                                      pre_relu=pre_relu)
        probs.append(prob.reshape(E, _DNP, _HEADS))

        agg = _sc_segmax(wgt, order_pad, ids_pad, bounds)
        sum_out = jax.ops.segment_sum(ue, row, num_segments=_N_NODES)
        sum_in = jax.ops.segment_sum(ue, col, num_segments=_N_NODES)
        zeros = jnp.zeros((_NP_PAD, _DIM), jnp.float32)

        fn = _node_compute(pad_n(nf), agg, pad_n(sum_out), zeros,
                           pad_n(sum_in), zeros, ico, ici, mask_out, wp)
        nf = fn[:_N_NODES]
        ef = ue
    return (nf, ef, probs)


# speedup vs baseline: 1.0826x; 1.0826x over previous
import functools

import jax
import jax.numpy as jnp
import numpy as np
from jax.experimental import pallas as pl
from jax.experimental.pallas import tpu as pltpu
from jax.experimental.pallas import tpu_sc as plsc

_N_NODES = 10000
_N_EDGES = 160000
_DIM = 128
_HEADS = 8
_DNP = _DIM // _HEADS
_TEMP = float(np.sqrt(_DNP))
_NP_PAD = 10240

_BE = 2000
_BN = 1024


_GW = 200


def _sc_gather(table, idx):
    n_orig = idx.shape[0]
    chunk = _GW * 32
    n = ((n_orig + chunk - 1) // chunk) * chunk
    if n != n_orig:
        idx = jnp.pad(idx, (0, n - n_orig))
    mesh = plsc.VectorSubcoreMesh(core_axis_name="core",
                                  subcore_axis_name="subcore")

    share = n // 32
    ncols = table.shape[1]

    @functools.partial(
        pl.kernel,
        out_type=jax.ShapeDtypeStruct((n, ncols), table.dtype),
        mesh=mesh,
        scratch_types=[pltpu.VMEM((_GW,), jnp.int32),
                       pltpu.VMEM((_GW, ncols), jnp.float32),
                       pltpu.SemaphoreType.DMA])
    def k(tab_hbm, i_hbm, o_hbm, idx_v, rows_v, sem):
        cid = jax.lax.axis_index("core")
        sid = jax.lax.axis_index("subcore")
        base = (cid * 16 + sid) * share

        @pl.loop(0, share, step=_GW)
        def _(off):
            pltpu.sync_copy(i_hbm.at[pl.ds(base + off, _GW)], idx_v)
            pltpu.async_copy(tab_hbm.at[idx_v], rows_v, sem).wait()
            pltpu.sync_copy(rows_v, o_hbm.at[pl.ds(base + off, _GW)])

    out = k(table, idx)
    return out[:n_orig] if n != n_orig else out


_RK = 104
_NPW = _NP_PAD // 32


def _sc_segreduce(values, values2, order_pad, ids_pad, bounds):
    mesh = plsc.VectorSubcoreMesh(core_axis_name="core",
                                  subcore_axis_name="subcore")
    neg = jnp.float32(-jnp.inf)
    out_t = jax.ShapeDtypeStruct((_NP_PAD, _DIM), jnp.float32)

    @functools.partial(
        pl.kernel,
        out_type=[out_t, out_t],
        mesh=mesh,
        scratch_types=[pltpu.VMEM((_NPW, _DIM), jnp.float32),
                       pltpu.VMEM((_NPW, _DIM), jnp.float32),
                       pltpu.VMEM((_RK,), jnp.int32),
                       pltpu.VMEM((_RK, _DIM), jnp.float32),
                       pltpu.VMEM((_RK, _DIM), jnp.float32),
                       pltpu.VMEM((_RK,), jnp.int32),
                       pltpu.VMEM((16,), jnp.int32),
                       pltpu.SemaphoreType.DMA])
    def k(val_hbm, val2_hbm, ord_hbm, ids_hbm, bnd_hbm, om_hbm, os_hbm,
          accm_v, accs_v, oidx_v, rows_v, rows2_v, ids_v, bnd_v, sem):
        cid = jax.lax.axis_index("core")
        sid = jax.lax.axis_index("subcore")
        w = cid * 16 + sid
        node_base = w * _NPW

        pltpu.sync_copy(bnd_hbm.at[w], bnd_v)
        bv = bnd_v[...]
        lo = bv[0]
        hi = bv[1]

        @pl.loop(0, _NPW)
        def _(r):
            @pl.loop(0, _DIM, step=16)
            def _(j):
                accm_v[r, pl.ds(j, 16)] = jnp.full((16,), neg)
                accs_v[r, pl.ds(j, 16)] = jnp.zeros((16,), jnp.float32)

        start0 = (lo // 8) * 8
        n_win = jax.lax.div(hi - start0 + _RK - 1, _RK)

        @pl.loop(0, n_win)
        def _(t):
            base = start0 + t * _RK
            pltpu.sync_copy(ord_hbm.at[pl.ds(base, _RK)], oidx_v)
            pltpu.sync_copy(ids_hbm.at[pl.ds(base, _RK)], ids_v)
            pltpu.async_copy(val_hbm.at[oidx_v], rows_v, sem).wait()
            pltpu.async_copy(val2_hbm.at[oidx_v], rows2_v, sem).wait()

            @pl.loop(0, _RK, step=16)
            def _(g):
                vec = ids_v[pl.ds(g, 16)]
                for e0 in range(16):
                    local = vec[e0] - node_base

                    @pl.when(jnp.logical_and(local >= 0, local < _NPW))
                    def _():
                        for j in range(0, _DIM, 16):
                            accm_v[local, pl.ds(j, 16)] = jnp.maximum(
                                accm_v[local, pl.ds(j, 16)],
                                rows_v[g + e0, pl.ds(j, 16)])
                            accs_v[local, pl.ds(j, 16)] = (
                                accs_v[local, pl.ds(j, 16)]
                                + rows2_v[g + e0, pl.ds(j, 16)])

        pltpu.sync_copy(accm_v, om_hbm.at[pl.ds(node_base, _NPW)])
        pltpu.sync_copy(accs_v, os_hbm.at[pl.ds(node_base, _NPW)])

    return k(values, values2, order_pad, ids_pad, bounds)


def _head_max(x):
    m = x
    for k in (8, 16, 32, 64):
        m = jnp.maximum(m, jnp.roll(m, k, axis=-1))
    return m


def _head_sum(x):
    s = x
    for k in (8, 16, 32, 64):
        s = s + jnp.roll(s, k, axis=-1)
    return s


def _edge_kernel(xi_ref, ef_ref, efr_ref, xj_ref, match_ref,
                 w1t_ref, b1_ref, w2t_ref, b2_ref,
                 wqt_ref, bq_ref, wkt_ref, bk_ref, wvt_ref, bv_ref,
                 m1_ref, a1_ref, m2_ref, a2_ref,
                 ue_ref, prob_ref, wgt_ref, *, pre_relu):
    xi = xi_ref[...]
    ef = ef_ref[...]
    efr = efr_ref[...] * match_ref[...]
    xj = xj_ref[...]
    if pre_relu:
        ef = jnp.maximum(ef, 0.0)
        efr = jnp.maximum(efr, 0.0)

    f32 = jnp.float32
    dot = functools.partial(jnp.dot, preferred_element_type=f32)

    w1t = w1t_ref[...]
    h = (dot(xi, w1t[0:128]) + dot(ef, w1t[128:256])
         + dot(efr, w1t[256:384]) + dot(xj, w1t[384:512]) + b1_ref[...])
    h = jnp.maximum(h, 0.0)
    ue = dot(h, w2t_ref[...]) + b2_ref[...]
    ue_ref[...] = ue

    q = dot(xi, wqt_ref[...]) + bq_ref[...]
    kk = dot(ef, wkt_ref[...]) + bk_ref[...]
    v = dot(xj, wvt_ref[...]) + bv_ref[...]

    m1 = m1_ref[...]
    a = dot(q, m1[0:128]) + dot(kk, m1[128:256]) + a1_ref[...]
    a = jnp.maximum(a, 0.0)
    att = dot(a, m2_ref[...]) + a2_ref[...]
    att = att * (1.0 / _TEMP)

    mx = _head_max(att)
    e = jnp.exp(att - mx)
    s = _head_sum(e)
    prob = e / s
    prob_ref[...] = prob
    wgt_ref[...] = prob * v


def _edge_compute(xi, ef, efr, xj, match, wp, pre_relu):
    grid = _N_EDGES // _BE
    bspec_e = pl.BlockSpec((_BE, _DIM), lambda i: (i, 0))
    bspec_m = pl.BlockSpec((_BE, 1), lambda i: (i, 0))

    def wspec(arr):
        return pl.BlockSpec(arr.shape, lambda i: tuple(0 for _ in arr.shape))

    weights = [wp['eu_W1T'], wp['eu_b1'], wp['eu_W2T'], wp['eu_b2'],
               wp['WqT'], wp['bq'], wp['WkT'], wp['bk'], wp['WvT'], wp['bv'],
               wp['M1'], wp['a1'], wp['M2'], wp['a2']]
    out_shape = [jax.ShapeDtypeStruct((_N_EDGES, _DIM), jnp.float32)] * 3
    return pl.pallas_call(
        functools.partial(_edge_kernel, pre_relu=pre_relu),
        grid=grid,
        in_specs=[bspec_e] * 4 + [bspec_m] + [wspec(w) for w in weights],
        out_specs=[bspec_e] * 3,
        out_shape=out_shape,
    )(xi, ef, efr, xj, match, *weights)


def _node_kernel(x_ref, agg_ref, so0_ref, so1_ref, si0_ref, si1_ref,
                 ico_ref, ici_ref, mask_ref,
                 nw1t_ref, nb1_ref, nw2t_ref, nb2_ref, eawt_ref, eab_ref,
                 out_ref):
    f32 = jnp.float32
    dot = functools.partial(jnp.dot, preferred_element_type=f32)
    x = x_ref[...]
    agg = agg_ref[...]
    agg = jnp.where(mask_ref[...] > 0.0, agg, 0.0)
    nw1t = nw1t_ref[...]
    h = dot(x, nw1t[0:128]) + dot(agg, nw1t[128:256]) + nb1_ref[...]
    h = jnp.maximum(h, 0.0)
    un = dot(h, nw2t_ref[...]) + nb2_ref[...]

    mean_out = (so0_ref[...] + so1_ref[...]) * ico_ref[...]
    mean_in = (si0_ref[...] + si1_ref[...]) * ici_ref[...]
    eawt = eawt_ref[...]
    logits = dot(mean_out, eawt[0:128]) + dot(mean_in, eawt[128:256]) + eab_ref[...]
    ea = jax.nn.sigmoid(logits)
    out_ref[...] = jnp.maximum(un, 0.0) * ea


def _node_compute(x_pad, agg, so0, so1, si0, si1, ico, ici, mask, wp):
    grid = _NP_PAD // _BN
    bspec_n = pl.BlockSpec((_BN, _DIM), lambda i: (i, 0))
    bspec_1 = pl.BlockSpec((_BN, 1), lambda i: (i, 0))

    def wspec(arr):
        return pl.BlockSpec(arr.shape, lambda i: tuple(0 for _ in arr.shape))

    weights = [wp['nu_W1T'], wp['nu_b1'], wp['nu_W2T'], wp['nu_b2'],
               wp['ea_WT'], wp['ea_b']]
    return pl.pallas_call(
        _node_kernel,
        grid=grid,
        in_specs=[bspec_n] * 6 + [bspec_1] * 3 + [wspec(w) for w in weights],
        out_specs=bspec_n,
        out_shape=jax.ShapeDtypeStruct((_NP_PAD, _DIM), jnp.float32),
    )(x_pad, agg, so0, so1, si0, si1, ico, ici, mask, *weights)


def _prep_weights(p):
    eye8 = jnp.eye(_HEADS, dtype=jnp.float32)
    return {
        'eu_W1T': p['eu_W1'].T, 'eu_b1': p['eu_b1'][None, :],
        'eu_W2T': p['eu_W2'].T, 'eu_b2': p['eu_b2'][None, :],
        'WqT': p['Wq'].T, 'bq': p['bq'][None, :],
        'WkT': p['Wk'].T, 'bk': p['bk'][None, :],
        'WvT': p['Wv'].T, 'bv': p['bv'][None, :],
        'M1': jnp.kron(p['att_W1'].T, eye8),
        'a1': jnp.repeat(p['att_b1'], _HEADS)[None, :],
        'M2': jnp.kron(p['att_W2'].T, eye8),
        'a2': jnp.repeat(p['att_b2'], _HEADS)[None, :],
        'nu_W1T': p['nu_W1'].T, 'nu_b1': p['nu_b1'][None, :],
        'nu_W2T': p['nu_W2'].T, 'nu_b2': p['nu_b2'][None, :],
        'ea_WT': p['ea_W'].T, 'ea_b': p['ea_b'][None, :],
    }


def kernel(x, edge_feature, edge_index, params):
    row, col = edge_index[0], edge_index[1]
    E = _N_EDGES

    keys = row * _N_NODES + col
    rev = col * _N_NODES + row
    order = jnp.argsort(keys)
    sk = keys[order]
    pos = jnp.clip(jnp.searchsorted(sk, rev), 0, E - 1)
    match = sk[pos] == rev
    rev_idx = jnp.where(match, order[pos], jnp.arange(E, dtype=pos.dtype))
    match_f = match.astype(jnp.float32)[:, None]

    ones = jnp.ones((E,), jnp.float32)
    cnt_out = jax.ops.segment_sum(ones, row, num_segments=_N_NODES)
    cnt_in = jax.ops.segment_sum(ones, col, num_segments=_N_NODES)

    row_sorted = (sk // _N_NODES).astype(jnp.int32)
    bnd = jnp.searchsorted(
        row_sorted, jnp.arange(33, dtype=jnp.int32) * _NPW).astype(jnp.int32)
    bounds = jnp.zeros((32, 16), jnp.int32)
    bounds = bounds.at[:, 0].set(bnd[:32]).at[:, 1].set(bnd[1:33])
    order_pad = jnp.pad(order.astype(jnp.int32), (0, _RK))
    ids_pad = jnp.pad(row_sorted, (0, _RK), constant_values=1 << 20)

    def pad_n(v):
        return jnp.pad(v, ((0, _NP_PAD - _N_NODES), (0, 0)))

    ico = pad_n((1.0 / jnp.maximum(cnt_out, 1.0))[:, None])
    ici = pad_n((1.0 / jnp.maximum(cnt_in, 1.0))[:, None])
    mask_out = pad_n((cnt_out > 0).astype(jnp.float32)[:, None])

    nf, ef = x, edge_feature
    probs = []
    for li, p in enumerate(params):
        wp = _prep_weights(p)
        pre_relu = li > 0
        xi = _sc_gather(nf, row)
        xj = _sc_gather(nf, col)
        efr = _sc_gather(ef, rev_idx)

        ue, prob, wgt = _edge_compute(xi, ef, efr, xj, match_f, wp,
                                      pre_relu=pre_relu)
        probs.append(prob.reshape(E, _DNP, _HEADS))

        agg, sum_out = _sc_segreduce(wgt, ue, order_pad, ids_pad, bounds)
        sum_in = jax.ops.segment_sum(ue, col, num_segments=_N_NODES)
        zeros = jnp.zeros((_NP_PAD, _DIM), jnp.float32)

        fn = _node_compute(pad_n(nf), agg, sum_out, zeros,
                           pad_n(sum_in), zeros, ico, ici, mask_out, wp)
        nf = fn[:_N_NODES]
        ef = ue
    return (nf, ef, probs)


# speedup vs baseline: 1.0903x; 1.0071x over previous
import functools

import jax
import jax.numpy as jnp
import numpy as np
from jax.experimental import pallas as pl
from jax.experimental.pallas import tpu as pltpu
from jax.experimental.pallas import tpu_sc as plsc

_N_NODES = 10000
_N_EDGES = 160000
_DIM = 128
_HEADS = 8
_DNP = _DIM // _HEADS
_TEMP = float(np.sqrt(_DNP))
_NP_PAD = 10240

_BE = 4000
_BN = 1024


_GW = 200


def _sc_gather(table, idx):
    n_orig = idx.shape[0]
    chunk = _GW * 32
    n = ((n_orig + chunk - 1) // chunk) * chunk
    if n != n_orig:
        idx = jnp.pad(idx, (0, n - n_orig))
    mesh = plsc.VectorSubcoreMesh(core_axis_name="core",
                                  subcore_axis_name="subcore")

    share = n // 32
    ncols = table.shape[1]

    @functools.partial(
        pl.kernel,
        out_type=jax.ShapeDtypeStruct((n, ncols), table.dtype),
        mesh=mesh,
        scratch_types=[pltpu.VMEM((_GW,), jnp.int32),
                       pltpu.VMEM((_GW, ncols), jnp.float32),
                       pltpu.SemaphoreType.DMA])
    def k(tab_hbm, i_hbm, o_hbm, idx_v, rows_v, sem):
        cid = jax.lax.axis_index("core")
        sid = jax.lax.axis_index("subcore")
        base = (cid * 16 + sid) * share

        @pl.loop(0, share, step=_GW)
        def _(off):
            pltpu.sync_copy(i_hbm.at[pl.ds(base + off, _GW)], idx_v)
            pltpu.async_copy(tab_hbm.at[idx_v], rows_v, sem).wait()
            pltpu.sync_copy(rows_v, o_hbm.at[pl.ds(base + off, _GW)])

    out = k(table, idx)
    return out[:n_orig] if n != n_orig else out


_RK = 104
_NPW = _NP_PAD // 32


def _sc_segreduce(values, values2, order_pad, ids_pad, bounds):
    mesh = plsc.VectorSubcoreMesh(core_axis_name="core",
                                  subcore_axis_name="subcore")
    neg = jnp.float32(-jnp.inf)
    out_t = jax.ShapeDtypeStruct((_NP_PAD, _DIM), jnp.float32)

    @functools.partial(
        pl.kernel,
        out_type=[out_t, out_t],
        mesh=mesh,
        scratch_types=[pltpu.VMEM((_NPW, _DIM), jnp.float32),
                       pltpu.VMEM((_NPW, _DIM), jnp.float32),
                       pltpu.VMEM((_RK,), jnp.int32),
                       pltpu.VMEM((_RK, _DIM), jnp.float32),
                       pltpu.VMEM((_RK, _DIM), jnp.float32),
                       pltpu.VMEM((_RK,), jnp.int32),
                       pltpu.VMEM((16,), jnp.int32),
                       pltpu.SemaphoreType.DMA])
    def k(val_hbm, val2_hbm, ord_hbm, ids_hbm, bnd_hbm, om_hbm, os_hbm,
          accm_v, accs_v, oidx_v, rows_v, rows2_v, ids_v, bnd_v, sem):
        cid = jax.lax.axis_index("core")
        sid = jax.lax.axis_index("subcore")
        w = cid * 16 + sid
        node_base = w * _NPW

        pltpu.sync_copy(bnd_hbm.at[w], bnd_v)
        bv = bnd_v[...]
        lo = bv[0]
        hi = bv[1]

        @pl.loop(0, _NPW)
        def _(r):
            @pl.loop(0, _DIM, step=16)
            def _(j):
                accm_v[r, pl.ds(j, 16)] = jnp.full((16,), neg)
                accs_v[r, pl.ds(j, 16)] = jnp.zeros((16,), jnp.float32)

        start0 = (lo // 8) * 8
        n_win = jax.lax.div(hi - start0 + _RK - 1, _RK)

        @pl.loop(0, n_win)
        def _(t):
            base = start0 + t * _RK
            pltpu.sync_copy(ord_hbm.at[pl.ds(base, _RK)], oidx_v)
            pltpu.sync_copy(ids_hbm.at[pl.ds(base, _RK)], ids_v)
            pltpu.async_copy(val_hbm.at[oidx_v], rows_v, sem).wait()
            pltpu.async_copy(val2_hbm.at[oidx_v], rows2_v, sem).wait()

            @pl.loop(0, _RK, step=16)
            def _(g):
                vec = ids_v[pl.ds(g, 16)]
                for e0 in range(16):
                    local = vec[e0] - node_base

                    @pl.when(jnp.logical_and(local >= 0, local < _NPW))
                    def _():
                        for j in range(0, _DIM, 16):
                            accm_v[local, pl.ds(j, 16)] = jnp.maximum(
                                accm_v[local, pl.ds(j, 16)],
                                rows_v[g + e0, pl.ds(j, 16)])
                            accs_v[local, pl.ds(j, 16)] = (
                                accs_v[local, pl.ds(j, 16)]
                                + rows2_v[g + e0, pl.ds(j, 16)])

        pltpu.sync_copy(accm_v, om_hbm.at[pl.ds(node_base, _NPW)])
        pltpu.sync_copy(accs_v, os_hbm.at[pl.ds(node_base, _NPW)])

    return k(values, values2, order_pad, ids_pad, bounds)


def _head_max(x):
    m = x
    for k in (8, 16, 32, 64):
        m = jnp.maximum(m, jnp.roll(m, k, axis=-1))
    return m


def _head_sum(x):
    s = x
    for k in (8, 16, 32, 64):
        s = s + jnp.roll(s, k, axis=-1)
    return s


def _edge_kernel(xi_ref, ef_ref, efr_ref, xj_ref, match_ref,
                 w1t_ref, b1_ref, w2t_ref, b2_ref,
                 wqt_ref, bq_ref, wkt_ref, bk_ref, wvt_ref, bv_ref,
                 m1_ref, a1_ref, m2_ref, a2_ref,
                 ue_ref, prob_ref, wgt_ref, *, pre_relu):
    xi = xi_ref[...]
    ef = ef_ref[...]
    efr = efr_ref[...] * match_ref[...]
    xj = xj_ref[...]
    if pre_relu:
        ef = jnp.maximum(ef, 0.0)
        efr = jnp.maximum(efr, 0.0)

    f32 = jnp.float32
    dot = functools.partial(jnp.dot, preferred_element_type=f32)

    w1t = w1t_ref[...]
    h = (dot(xi, w1t[0:128]) + dot(ef, w1t[128:256])
         + dot(efr, w1t[256:384]) + dot(xj, w1t[384:512]) + b1_ref[...])
    h = jnp.maximum(h, 0.0)
    ue = dot(h, w2t_ref[...]) + b2_ref[...]
    ue_ref[...] = ue

    q = dot(xi, wqt_ref[...]) + bq_ref[...]
    kk = dot(ef, wkt_ref[...]) + bk_ref[...]
    v = dot(xj, wvt_ref[...]) + bv_ref[...]

    m1 = m1_ref[...]
    a = dot(q, m1[0:128]) + dot(kk, m1[128:256]) + a1_ref[...]
    a = jnp.maximum(a, 0.0)
    att = dot(a, m2_ref[...]) + a2_ref[...]
    att = att * (1.0 / _TEMP)

    mx = _head_max(att)
    e = jnp.exp(att - mx)
    s = _head_sum(e)
    prob = e / s
    prob_ref[...] = prob
    wgt_ref[...] = prob * v


def _edge_compute(xi, ef, efr, xj, match, wp, pre_relu):
    grid = _N_EDGES // _BE
    bspec_e = pl.BlockSpec((_BE, _DIM), lambda i: (i, 0))
    bspec_m = pl.BlockSpec((_BE, 1), lambda i: (i, 0))

    def wspec(arr):
        return pl.BlockSpec(arr.shape, lambda i: tuple(0 for _ in arr.shape))

    weights = [wp['eu_W1T'], wp['eu_b1'], wp['eu_W2T'], wp['eu_b2'],
               wp['WqT'], wp['bq'], wp['WkT'], wp['bk'], wp['WvT'], wp['bv'],
               wp['M1'], wp['a1'], wp['M2'], wp['a2']]
    out_shape = [jax.ShapeDtypeStruct((_N_EDGES, _DIM), jnp.float32)] * 3
    return pl.pallas_call(
        functools.partial(_edge_kernel, pre_relu=pre_relu),
        grid=grid,
        in_specs=[bspec_e] * 4 + [bspec_m] + [wspec(w) for w in weights],
        out_specs=[bspec_e] * 3,
        out_shape=out_shape,
    )(xi, ef, efr, xj, match, *weights)


def _node_kernel(x_ref, agg_ref, so0_ref, so1_ref, si0_ref, si1_ref,
                 ico_ref, ici_ref, mask_ref,
                 nw1t_ref, nb1_ref, nw2t_ref, nb2_ref, eawt_ref, eab_ref,
                 out_ref):
    f32 = jnp.float32
    dot = functools.partial(jnp.dot, preferred_element_type=f32)
    x = x_ref[...]
    agg = agg_ref[...]
    agg = jnp.where(mask_ref[...] > 0.0, agg, 0.0)
    nw1t = nw1t_ref[...]
    h = dot(x, nw1t[0:128]) + dot(agg, nw1t[128:256]) + nb1_ref[...]
    h = jnp.maximum(h, 0.0)
    un = dot(h, nw2t_ref[...]) + nb2_ref[...]

    mean_out = (so0_ref[...] + so1_ref[...]) * ico_ref[...]
    mean_in = (si0_ref[...] + si1_ref[...]) * ici_ref[...]
    eawt = eawt_ref[...]
    logits = dot(mean_out, eawt[0:128]) + dot(mean_in, eawt[128:256]) + eab_ref[...]
    ea = jax.nn.sigmoid(logits)
    out_ref[...] = jnp.maximum(un, 0.0) * ea


def _node_compute(x_pad, agg, so0, so1, si0, si1, ico, ici, mask, wp):
    grid = _NP_PAD // _BN
    bspec_n = pl.BlockSpec((_BN, _DIM), lambda i: (i, 0))
    bspec_1 = pl.BlockSpec((_BN, 1), lambda i: (i, 0))

    def wspec(arr):
        return pl.BlockSpec(arr.shape, lambda i: tuple(0 for _ in arr.shape))

    weights = [wp['nu_W1T'], wp['nu_b1'], wp['nu_W2T'], wp['nu_b2'],
               wp['ea_WT'], wp['ea_b']]
    return pl.pallas_call(
        _node_kernel,
        grid=grid,
        in_specs=[bspec_n] * 6 + [bspec_1] * 3 + [wspec(w) for w in weights],
        out_specs=bspec_n,
        out_shape=jax.ShapeDtypeStruct((_NP_PAD, _DIM), jnp.float32),
    )(x_pad, agg, so0, so1, si0, si1, ico, ici, mask, *weights)


def _prep_weights(p):
    eye8 = jnp.eye(_HEADS, dtype=jnp.float32)
    return {
        'eu_W1T': p['eu_W1'].T, 'eu_b1': p['eu_b1'][None, :],
        'eu_W2T': p['eu_W2'].T, 'eu_b2': p['eu_b2'][None, :],
        'WqT': p['Wq'].T, 'bq': p['bq'][None, :],
        'WkT': p['Wk'].T, 'bk': p['bk'][None, :],
        'WvT': p['Wv'].T, 'bv': p['bv'][None, :],
        'M1': jnp.kron(p['att_W1'].T, eye8),
        'a1': jnp.repeat(p['att_b1'], _HEADS)[None, :],
        'M2': jnp.kron(p['att_W2'].T, eye8),
        'a2': jnp.repeat(p['att_b2'], _HEADS)[None, :],
        'nu_W1T': p['nu_W1'].T, 'nu_b1': p['nu_b1'][None, :],
        'nu_W2T': p['nu_W2'].T, 'nu_b2': p['nu_b2'][None, :],
        'ea_WT': p['ea_W'].T, 'ea_b': p['ea_b'][None, :],
    }


def kernel(x, edge_feature, edge_index, params):
    row, col = edge_index[0], edge_index[1]
    E = _N_EDGES

    keys = row * _N_NODES + col
    rev = col * _N_NODES + row
    order = jnp.argsort(keys)
    sk = keys[order]
    pos = jnp.clip(jnp.searchsorted(sk, rev), 0, E - 1)
    match = sk[pos] == rev
    rev_idx = jnp.where(match, order[pos], jnp.arange(E, dtype=pos.dtype))
    match_f = match.astype(jnp.float32)[:, None]

    ones = jnp.ones((E,), jnp.float32)
    cnt_out = jax.ops.segment_sum(ones, row, num_segments=_N_NODES)
    cnt_in = jax.ops.segment_sum(ones, col, num_segments=_N_NODES)

    row_sorted = (sk // _N_NODES).astype(jnp.int32)
    bnd = jnp.searchsorted(
        row_sorted, jnp.arange(33, dtype=jnp.int32) * _NPW).astype(jnp.int32)
    bounds = jnp.zeros((32, 16), jnp.int32)
    bounds = bounds.at[:, 0].set(bnd[:32]).at[:, 1].set(bnd[1:33])
    order_pad = jnp.pad(order.astype(jnp.int32), (0, _RK))
    ids_pad = jnp.pad(row_sorted, (0, _RK), constant_values=1 << 20)

    def pad_n(v):
        return jnp.pad(v, ((0, _NP_PAD - _N_NODES), (0, 0)))

    ico = pad_n((1.0 / jnp.maximum(cnt_out, 1.0))[:, None])
    ici = pad_n((1.0 / jnp.maximum(cnt_in, 1.0))[:, None])
    mask_out = pad_n((cnt_out > 0).astype(jnp.float32)[:, None])

    nf, ef = x, edge_feature
    probs = []
    for li, p in enumerate(params):
        wp = _prep_weights(p)
        pre_relu = li > 0
        xi = _sc_gather(nf, row)
        xj = _sc_gather(nf, col)
        efr = _sc_gather(ef, rev_idx)

        ue, prob, wgt = _edge_compute(xi, ef, efr, xj, match_f, wp,
                                      pre_relu=pre_relu)
        probs.append(prob.reshape(E, _DNP, _HEADS))

        agg, sum_out = _sc_segreduce(wgt, ue, order_pad, ids_pad, bounds)
        sum_in = jax.ops.segment_sum(ue, col, num_segments=_N_NODES)
        zeros = jnp.zeros((_NP_PAD, _DIM), jnp.float32)

        fn = _node_compute(pad_n(nf), agg, sum_out, zeros,
                           pad_n(sum_in), zeros, ico, ici, mask_out, wp)
        nf = fn[:_N_NODES]
        ef = ue
    return (nf, ef, probs)
